# baseline (speedup 1.0000x reference)
import functools

import jax
import jax.numpy as jnp
from jax import lax
from jax.experimental import pallas as pl
from jax.experimental.pallas import tpu as pltpu
from jax.experimental.pallas import tpu_sc as plsc

EMB_COUNT = 1000000
EMB_DIM = 32
NEG_K = 20
CTX_LEN = 20
BATCH_N = 16384

NUM_CORES = 2
NUM_SUBCORES = 16
NW = NUM_CORES * NUM_SUBCORES
BPW = BATCH_N // NW
CB = 64
NCHUNK = BPW // CB
CROWS = CB * CTX_LEN


TW = 512
NFULL = EMB_COUNT // TW
TAIL = EMB_COUNT - NFULL * TW


def _sc_detile(ot_t, it_t, o_tail, i_tail):
    mesh = plsc.VectorSubcoreMesh(core_axis_name="c", subcore_axis_name="s")

    @functools.partial(
        pl.kernel,
        mesh=mesh,
        compiler_params=pltpu.CompilerParams(
            use_tc_tiling_on_sc=True, needs_layout_passes=False),
        out_type=(
            jax.ShapeDtypeStruct((EMB_COUNT * EMB_DIM,), jnp.float32),
            jax.ShapeDtypeStruct((EMB_COUNT * EMB_DIM,), jnp.float32),
        ),
        scratch_types=[
            pltpu.VMEM((EMB_DIM, TW), jnp.float32),
            pltpu.VMEM((TW * EMB_DIM,), jnp.float32),
            pltpu.SemaphoreType.DMA,
        ],
    )
    def k(ot_hbm, it_hbm, otail_hbm, itail_hbm, oflat_hbm, iflat_hbm,
          in_v, out_v, sem):
        wid = lax.axis_index("s") * NUM_CORES + lax.axis_index("c")
        lane = lax.iota(jnp.int32, 16)
        d_lo = lane
        d_hi = lane + 16

        def transpose_groups(ngroups):
            def gbody(g, carry):
                r0 = g * 16
                for m in range(32):
                    ridx = jnp.full((16,), r0 + (m // 2), jnp.int32)
                    v = plsc.load_gather(
                        in_v, [d_lo if m % 2 == 0 else d_hi, ridx])
                    out_v[pl.ds(r0 * EMB_DIM + m * 16, 16)] = v
                return carry
            lax.fori_loop(0, ngroups, gbody, 0)

        for src, tail, dst in ((ot_hbm, otail_hbm, oflat_hbm),
                               (it_hbm, itail_hbm, iflat_hbm)):
            nchunk = (NFULL - 1 - wid) // NW + 1

            def cbody(j, carry):
                c = wid + j * NW
                for kk in range(EMB_DIM // 8):
                    pltpu.sync_copy(
                        src.at[pl.ds(kk * 8, 8), pl.ds(c * TW, TW)],
                        in_v.at[pl.ds(kk * 8, 8), :])
                transpose_groups(TW // 16)
                pltpu.sync_copy(
                    out_v, dst.at[pl.ds(c * TW * EMB_DIM, TW * EMB_DIM)])
                return carry

            lax.fori_loop(0, nchunk, cbody, 0)

            @pl.when(wid == NW - 1)
            def _():
                def tail_inner(t_v):
                    pltpu.sync_copy(tail, t_v)

                    def gbody(g, carry):
                        r0 = g * 16
                        for m in range(32):
                            dv = d_lo if m % 2 == 0 else d_hi
                            idx = dv * TAIL + (r0 + (m // 2))
                            v = plsc.load_gather(t_v, [idx])
                            out_v[pl.ds(r0 * EMB_DIM + m * 16, 16)] = v
                        return carry

                    lax.fori_loop(0, TAIL // 16, gbody, 0)
                    pltpu.sync_copy(
                        out_v.at[pl.ds(0, TAIL * EMB_DIM)],
                        dst.at[pl.ds(NFULL * TW * EMB_DIM, TAIL * EMB_DIM)])

                pl.run_scoped(
                    tail_inner,
                    pltpu.VMEM((EMB_DIM * TAIL,), jnp.float32))

    return k(ot_t, it_t, o_tail, i_tail)


def _sc_scores(ctx_idx, neg_idx, tgt_idx, i_table, o_table):
    mesh = plsc.VectorSubcoreMesh(core_axis_name="c", subcore_axis_name="s")

    @functools.partial(
        pl.kernel,
        mesh=mesh,
        compiler_params=pltpu.CompilerParams(
            use_tc_tiling_on_sc=False, needs_layout_passes=False),
        out_type=(
            jax.ShapeDtypeStruct((BATCH_N * CTX_LEN,), jnp.float32),
            jax.ShapeDtypeStruct((BATCH_N * NEG_K,), jnp.float32),
        ),
        scratch_types=[
            pltpu.VMEM((CROWS,), jnp.int32),
            pltpu.VMEM((CROWS,), jnp.int32),
            pltpu.VMEM((CB,), jnp.int32),
            pltpu.VMEM((CROWS, EMB_DIM), jnp.float32),
            pltpu.VMEM((CROWS, EMB_DIM), jnp.float32),
            pltpu.VMEM((CB, EMB_DIM), jnp.float32),
            pltpu.VMEM((CROWS,), jnp.float32),
            pltpu.VMEM((CROWS,), jnp.float32),
            pltpu.SemaphoreType.DMA,
        ],
    )
    def k(ctx_hbm, neg_hbm, tgt_hbm, it_hbm, ot_hbm, ps_hbm, ns_hbm,
          cidx_v, nidx_v, tidx_v, crows_v, nrows_v, trows_v, ps_v, ns_v, sem):
        wid = lax.axis_index("s") * NUM_CORES + lax.axis_index("c")
        lane = lax.iota(jnp.int32, 16)
        masks = [lane == l for l in range(16)]

        def chunk_body(t, carry0):
            roff = (wid * NCHUNK + t) * CROWS
            boff = (wid * NCHUNK + t) * CB
            pltpu.sync_copy(ctx_hbm.at[pl.ds(roff, CROWS)], cidx_v)
            pltpu.sync_copy(neg_hbm.at[pl.ds(roff, CROWS)], nidx_v)
            pltpu.sync_copy(tgt_hbm.at[pl.ds(boff, CB)], tidx_v)
            g1 = pltpu.async_copy(ot_hbm.at[cidx_v], crows_v, sem)
            g2 = pltpu.async_copy(ot_hbm.at[nidx_v], nrows_v, sem)
            g3 = pltpu.async_copy(it_hbm.at[tidx_v], trows_v, sem)
            g1.wait()
            g2.wait()
            g3.wait()

            def group_body(g, carry1):
                t0 = t1 = None
                pacc = nacc = None
                for j in range(4 * CTX_LEN):
                    if j % CTX_LEN == 0:
                        b = g * 4 + (j // CTX_LEN)
                        t0 = trows_v[b, pl.ds(0, 16)]
                        t1 = trows_v[b, pl.ds(16, 16)]
                    i = g * (4 * CTX_LEN) + j
                    v, l = j // 16, j % 16
                    p = (crows_v[i, pl.ds(0, 16)] * t0
                         + crows_v[i, pl.ds(16, 16)] * t1)
                    q = (nrows_v[i, pl.ds(0, 16)] * t0
                         + nrows_v[i, pl.ds(16, 16)] * t1)
                    ps = jnp.sum(p)
                    ns = jnp.sum(q)
                    if l == 0:
                        pacc = jnp.where(masks[0], ps, 0.0)
                        nacc = jnp.where(masks[0], ns, 0.0)
                    else:
                        pacc = jnp.where(masks[l], ps, pacc)
                        nacc = jnp.where(masks[l], ns, nacc)
                    if l == 15:
                        off = g * (4 * CTX_LEN) + v * 16
                        ps_v[pl.ds(off, 16)] = pacc
                        ns_v[pl.ds(off, 16)] = nacc
                return carry1

            lax.fori_loop(0, CROWS // (4 * CTX_LEN), group_body, 0)
            pltpu.sync_copy(ps_v, ps_hbm.at[pl.ds(roff, CROWS)])
            pltpu.sync_copy(ns_v, ns_hbm.at[pl.ds(roff, CROWS)])
            return carry0

        lax.fori_loop(0, NCHUNK, chunk_body, 0)

    return k(ctx_idx, neg_idx, tgt_idx, i_table, o_table)


def _log_sigmoid(x):
    return jnp.minimum(x, 0.0) - jnp.log1p(jnp.exp(-jnp.abs(x)))


def _tc_loss_body(ps_ref, ns_ref, out_ref):
    out_ref[0] = jnp.sum(_log_sigmoid(ps_ref[...]))
    out_ref[1] = jnp.sum(_log_sigmoid(-ns_ref[...]))


def _tc_loss(pos_s, neg_s):
    n = BATCH_N * CTX_LEN
    acc = pl.pallas_call(
        _tc_loss_body,
        in_specs=[
            pl.BlockSpec((n // 128, 128), lambda: (0, 0)),
            pl.BlockSpec((n // 128, 128), lambda: (0, 0)),
        ],
        out_specs=pl.BlockSpec(memory_space=pltpu.SMEM),
        out_shape=jax.ShapeDtypeStruct((2,), jnp.float32),
    )(pos_s.reshape(n // 128, 128), neg_s.reshape(n // 128, 128))
    return -(acc[0] / CTX_LEN + acc[1]) / BATCH_N


def kernel(context, target, i_table, o_table):
    b = context.shape[0]
    neg_samples = jax.random.randint(
        jax.random.key(12345), (b, NEG_K), 0, EMB_COUNT - 1)
    ctx_idx = context.astype(jnp.int32).reshape(-1)
    neg_idx = neg_samples.astype(jnp.int32).reshape(-1)
    tgt_idx = target.astype(jnp.int32)
    o_tail = o_table.T[:, NFULL * TW:].reshape(-1)
    i_tail = i_table.T[:, NFULL * TW:].reshape(-1)
    oflat, iflat = _sc_detile(o_table.T, i_table.T, o_tail, i_tail)
    o_c = oflat.reshape(EMB_COUNT, EMB_DIM)
    i_c = iflat.reshape(EMB_COUNT, EMB_DIM)
    pos_s, neg_s = _sc_scores(ctx_idx, neg_idx, tgt_idx, i_c, o_c)
    return _tc_loss(pos_s, neg_s)


# speedup vs baseline: 1.2924x; 1.2924x over previous
import functools

import jax
import jax.numpy as jnp
from jax import lax
from jax.experimental import pallas as pl
from jax.experimental.pallas import tpu as pltpu
from jax.experimental.pallas import tpu_sc as plsc

EMB_COUNT = 1000000
EMB_DIM = 32
NEG_K = 20
CTX_LEN = 20
BATCH_N = 16384

NUM_CORES = 2
NUM_SUBCORES = 16
NW = NUM_CORES * NUM_SUBCORES
BPW = BATCH_N // NW
CB = 64
NCHUNK = BPW // CB
CROWS = CB * CTX_LEN


TW = 768
NFULL = EMB_COUNT // TW
TAIL = EMB_COUNT - NFULL * TW
NCH = (NFULL + NW - 1) // NW


def _sc_detile(ot_t, it_t, o_tail, i_tail):
    mesh = plsc.VectorSubcoreMesh(core_axis_name="c", subcore_axis_name="s")

    @functools.partial(
        pl.kernel,
        mesh=mesh,
        compiler_params=pltpu.CompilerParams(
            use_tc_tiling_on_sc=True, needs_layout_passes=False),
        out_type=(
            jax.ShapeDtypeStruct((EMB_COUNT * EMB_DIM,), jnp.float32),
            jax.ShapeDtypeStruct((EMB_COUNT * EMB_DIM,), jnp.float32),
        ),
        scratch_types=[
            pltpu.VMEM((EMB_DIM, TW), jnp.float32),
            pltpu.VMEM((EMB_DIM, TW), jnp.float32),
            pltpu.VMEM((TW * EMB_DIM,), jnp.float32),
            pltpu.VMEM((TW * EMB_DIM,), jnp.float32),
            pltpu.SemaphoreType.DMA,
            pltpu.SemaphoreType.DMA,
            pltpu.SemaphoreType.DMA,
            pltpu.SemaphoreType.DMA,
        ],
    )
    def k(ot_hbm, it_hbm, otail_hbm, itail_hbm, oflat_hbm, iflat_hbm,
          in_v0, in_v1, out_v0, out_v1, s_i0, s_i1, s_o0, s_o1):
        wid = lax.axis_index("s") * NUM_CORES + lax.axis_index("c")
        lane = lax.iota(jnp.int32, 16)
        d_lo = lane
        d_hi = lane + 16
        ins = (in_v0, in_v1)
        outs = (out_v0, out_v1)
        s_in = (s_i0, s_i1)
        s_out = (s_o0, s_o1)
        out_v = out_v0

        def transpose_groups(in_ref, out_ref, ngroups):
            def gbody(g, carry):
                r0 = g * 16
                for m in range(32):
                    ridx = jnp.full((16,), r0 + (m // 2), jnp.int32)
                    v = plsc.load_gather(
                        in_ref, [d_lo if m % 2 == 0 else d_hi, ridx])
                    out_ref[pl.ds(r0 * EMB_DIM + m * 16, 16)] = v
                return carry
            lax.fori_loop(0, ngroups, gbody, 0)

        jmax = (NFULL - 1 - wid) // NW

        for src, tail, dst in ((ot_hbm, otail_hbm, oflat_hbm),
                               (it_hbm, itail_hbm, iflat_hbm)):

            def in_args(c, b, kk):
                return (src.at[pl.ds(kk * 8, 8), pl.ds(c * TW, TW)],
                        ins[b].at[pl.ds(kk * 8, 8), :], s_in[b])

            def out_args(c, b):
                return (outs[b],
                        dst.at[pl.ds(c * TW * EMB_DIM, TW * EMB_DIM)],
                        s_out[b])

            def issue_in(c, b):
                for kk in range(EMB_DIM // 8):
                    pltpu.async_copy(*in_args(c, b, kk))

            def wait_in(c, b):
                for kk in range(EMB_DIM // 8):
                    pltpu.make_async_copy(*in_args(c, b, kk)).wait()

            issue_in(wid, 0)

            def pair_body(jj, carry):
                for b in (0, 1):
                    j = jj * 2 + b
                    c = wid + j * NW

                    @pl.when(c < NFULL)
                    def _():
                        @pl.when(c + NW < NFULL)
                        def _():
                            issue_in(c + NW, 1 - b)
                        wait_in(c, b)

                        @pl.when(j >= 2)
                        def _():
                            pltpu.make_async_copy(
                                *out_args(c - 2 * NW, b)).wait()
                        transpose_groups(ins[b], outs[b], TW // 16)
                        pltpu.async_copy(*out_args(c, b))
                return carry

            lax.fori_loop(0, (NCH + 1) // 2, pair_body, 0)

            for b_ in (0, 1):
                j_b = jmax - ((jmax - b_) % 2)

                @pl.when(j_b >= 0)
                def _():
                    pltpu.make_async_copy(
                        *out_args(wid + j_b * NW, b_)).wait()

            @pl.when(wid == NW - 1)
            def _():
                def tail_inner(t_v):
                    pltpu.sync_copy(tail, t_v)

                    def gbody(g, carry):
                        r0 = g * 16
                        for m in range(32):
                            dv = d_lo if m % 2 == 0 else d_hi
                            idx = dv * TAIL + (r0 + (m // 2))
                            v = plsc.load_gather(t_v, [idx])
                            out_v[pl.ds(r0 * EMB_DIM + m * 16, 16)] = v
                        return carry

                    lax.fori_loop(0, TAIL // 16, gbody, 0)
                    pltpu.sync_copy(
                        out_v.at[pl.ds(0, TAIL * EMB_DIM)],
                        dst.at[pl.ds(NFULL * TW * EMB_DIM, TAIL * EMB_DIM)])

                pl.run_scoped(
                    tail_inner,
                    pltpu.VMEM((EMB_DIM * TAIL,), jnp.float32))

    return k(ot_t, it_t, o_tail, i_tail)


def _sc_scores(ctx_idx, neg_idx, tgt_idx, i_table, o_table):
    mesh = plsc.VectorSubcoreMesh(core_axis_name="c", subcore_axis_name="s")

    @functools.partial(
        pl.kernel,
        mesh=mesh,
        compiler_params=pltpu.CompilerParams(
            use_tc_tiling_on_sc=False, needs_layout_passes=False),
        out_type=(
            jax.ShapeDtypeStruct((BATCH_N * CTX_LEN,), jnp.float32),
            jax.ShapeDtypeStruct((BATCH_N * NEG_K,), jnp.float32),
        ),
        scratch_types=[
            pltpu.VMEM((CROWS,), jnp.int32),
            pltpu.VMEM((CROWS,), jnp.int32),
            pltpu.VMEM((CB,), jnp.int32),
            pltpu.VMEM((CROWS, EMB_DIM), jnp.float32),
            pltpu.VMEM((CROWS, EMB_DIM), jnp.float32),
            pltpu.VMEM((CB, EMB_DIM), jnp.float32),
            pltpu.VMEM((CROWS,), jnp.float32),
            pltpu.VMEM((CROWS,), jnp.float32),
            pltpu.SemaphoreType.DMA,
        ],
    )
    def k(ctx_hbm, neg_hbm, tgt_hbm, it_hbm, ot_hbm, ps_hbm, ns_hbm,
          cidx_v, nidx_v, tidx_v, crows_v, nrows_v, trows_v, ps_v, ns_v, sem):
        wid = lax.axis_index("s") * NUM_CORES + lax.axis_index("c")
        lane = lax.iota(jnp.int32, 16)
        masks = [lane == l for l in range(16)]

        def chunk_body(t, carry0):
            roff = (wid * NCHUNK + t) * CROWS
            boff = (wid * NCHUNK + t) * CB
            pltpu.sync_copy(ctx_hbm.at[pl.ds(roff, CROWS)], cidx_v)
            pltpu.sync_copy(neg_hbm.at[pl.ds(roff, CROWS)], nidx_v)
            pltpu.sync_copy(tgt_hbm.at[pl.ds(boff, CB)], tidx_v)
            g1 = pltpu.async_copy(ot_hbm.at[cidx_v], crows_v, sem)
            g2 = pltpu.async_copy(ot_hbm.at[nidx_v], nrows_v, sem)
            g3 = pltpu.async_copy(it_hbm.at[tidx_v], trows_v, sem)
            g1.wait()
            g2.wait()
            g3.wait()

            def group_body(g, carry1):
                t0 = t1 = None
                pacc = nacc = None
                for j in range(4 * CTX_LEN):
                    if j % CTX_LEN == 0:
                        b = g * 4 + (j // CTX_LEN)
                        t0 = trows_v[b, pl.ds(0, 16)]
                        t1 = trows_v[b, pl.ds(16, 16)]
                    i = g * (4 * CTX_LEN) + j
                    v, l = j // 16, j % 16
                    p = (crows_v[i, pl.ds(0, 16)] * t0
                         + crows_v[i, pl.ds(16, 16)] * t1)
                    q = (nrows_v[i, pl.ds(0, 16)] * t0
                         + nrows_v[i, pl.ds(16, 16)] * t1)
                    ps = jnp.sum(p)
                    ns = jnp.sum(q)
                    if l == 0:
                        pacc = jnp.where(masks[0], ps, 0.0)
                        nacc = jnp.where(masks[0], ns, 0.0)
                    else:
                        pacc = jnp.where(masks[l], ps, pacc)
                        nacc = jnp.where(masks[l], ns, nacc)
                    if l == 15:
                        off = g * (4 * CTX_LEN) + v * 16
                        ps_v[pl.ds(off, 16)] = pacc
                        ns_v[pl.ds(off, 16)] = nacc
                return carry1

            lax.fori_loop(0, CROWS // (4 * CTX_LEN), group_body, 0)
            pltpu.sync_copy(ps_v, ps_hbm.at[pl.ds(roff, CROWS)])
            pltpu.sync_copy(ns_v, ns_hbm.at[pl.ds(roff, CROWS)])
            return carry0

        lax.fori_loop(0, NCHUNK, chunk_body, 0)

    return k(ctx_idx, neg_idx, tgt_idx, i_table, o_table)


def _log_sigmoid(x):
    return jnp.minimum(x, 0.0) - jnp.log1p(jnp.exp(-jnp.abs(x)))


def _tc_loss_body(ps_ref, ns_ref, out_ref):
    out_ref[0] = jnp.sum(_log_sigmoid(ps_ref[...]))
    out_ref[1] = jnp.sum(_log_sigmoid(-ns_ref[...]))


def _tc_loss(pos_s, neg_s):
    n = BATCH_N * CTX_LEN
    acc = pl.pallas_call(
        _tc_loss_body,
        in_specs=[
            pl.BlockSpec((n // 128, 128), lambda: (0, 0)),
            pl.BlockSpec((n // 128, 128), lambda: (0, 0)),
        ],
        out_specs=pl.BlockSpec(memory_space=pltpu.SMEM),
        out_shape=jax.ShapeDtypeStruct((2,), jnp.float32),
    )(pos_s.reshape(n // 128, 128), neg_s.reshape(n // 128, 128))
    return -(acc[0] / CTX_LEN + acc[1]) / BATCH_N


def kernel(context, target, i_table, o_table):
    b = context.shape[0]
    neg_samples = jax.random.randint(
        jax.random.key(12345), (b, NEG_K), 0, EMB_COUNT - 1)
    ctx_idx = context.astype(jnp.int32).reshape(-1)
    neg_idx = neg_samples.astype(jnp.int32).reshape(-1)
    tgt_idx = target.astype(jnp.int32)
    o_tail = o_table.T[:, NFULL * TW:].reshape(-1)
    i_tail = i_table.T[:, NFULL * TW:].reshape(-1)
    oflat, iflat = _sc_detile(o_table.T, i_table.T, o_tail, i_tail)
    o_c = oflat.reshape(EMB_COUNT, EMB_DIM)
    i_c = iflat.reshape(EMB_COUNT, EMB_DIM)
    pos_s, neg_s = _sc_scores(ctx_idx, neg_idx, tgt_idx, i_c, o_c)
    return _tc_loss(pos_s, neg_s)


# speedup vs baseline: 1.5705x; 1.2152x over previous
import functools

import jax
import jax.numpy as jnp
from jax import lax
from jax.experimental import pallas as pl
from jax.experimental.pallas import tpu as pltpu
from jax.experimental.pallas import tpu_sc as plsc

EMB_COUNT = 1000000
EMB_DIM = 32
NEG_K = 20
CTX_LEN = 20
BATCH_N = 16384

NUM_CORES = 2
NUM_SUBCORES = 16
NW = NUM_CORES * NUM_SUBCORES
BPW = BATCH_N // NW
CB = 64
NCHUNK = BPW // CB
CROWS = CB * CTX_LEN


TW = 768
NFULL = EMB_COUNT // TW
TAIL = EMB_COUNT - NFULL * TW
NCH = (NFULL + NW - 1) // NW


def _sc_detile(ot_t, it_t, o_tail, i_tail):
    mesh = plsc.VectorSubcoreMesh(core_axis_name="c", subcore_axis_name="s")

    @functools.partial(
        pl.kernel,
        mesh=mesh,
        compiler_params=pltpu.CompilerParams(
            use_tc_tiling_on_sc=True, needs_layout_passes=False),
        out_type=(
            jax.ShapeDtypeStruct((EMB_COUNT * EMB_DIM,), jnp.float32),
            jax.ShapeDtypeStruct((EMB_COUNT * EMB_DIM,), jnp.float32),
        ),
        scratch_types=[
            pltpu.VMEM((EMB_DIM, TW), jnp.float32),
            pltpu.VMEM((EMB_DIM, TW), jnp.float32),
            pltpu.VMEM((TW * EMB_DIM,), jnp.float32),
            pltpu.VMEM((TW * EMB_DIM,), jnp.float32),
            pltpu.SemaphoreType.DMA,
            pltpu.SemaphoreType.DMA,
            pltpu.SemaphoreType.DMA,
            pltpu.SemaphoreType.DMA,
        ],
    )
    def k(ot_hbm, it_hbm, otail_hbm, itail_hbm, oflat_hbm, iflat_hbm,
          in_v0, in_v1, out_v0, out_v1, s_i0, s_i1, s_o0, s_o1):
        wid = lax.axis_index("s") * NUM_CORES + lax.axis_index("c")
        lane = lax.iota(jnp.int32, 16)
        d_lo = lane
        d_hi = lane + 16
        ins = (in_v0, in_v1)
        outs = (out_v0, out_v1)
        s_in = (s_i0, s_i1)
        s_out = (s_o0, s_o1)
        out_v = out_v0

        lane32 = lane * EMB_DIM

        def transpose_groups(in_ref, out_ref, ngroups):
            def gbody(g, carry):
                r0 = g * 16
                base = lane32 + r0 * EMB_DIM
                for d in range(EMB_DIM):
                    v = in_ref[d, pl.ds(r0, 16)]
                    plsc.store_scatter(out_ref, [base + d], v)
                return carry
            lax.fori_loop(0, ngroups, gbody, 0)

        jmax = (NFULL - 1 - wid) // NW

        for src, tail, dst in ((ot_hbm, otail_hbm, oflat_hbm),
                               (it_hbm, itail_hbm, iflat_hbm)):

            def in_args(c, b, kk):
                return (src.at[pl.ds(kk * 8, 8), pl.ds(c * TW, TW)],
                        ins[b].at[pl.ds(kk * 8, 8), :], s_in[b])

            def out_args(c, b):
                return (outs[b],
                        dst.at[pl.ds(c * TW * EMB_DIM, TW * EMB_DIM)],
                        s_out[b])

            def issue_in(c, b):
                for kk in range(EMB_DIM // 8):
                    pltpu.async_copy(*in_args(c, b, kk))

            def wait_in(c, b):
                for kk in range(EMB_DIM // 8):
                    pltpu.make_async_copy(*in_args(c, b, kk)).wait()

            issue_in(wid, 0)

            def pair_body(jj, carry):
                for b in (0, 1):
                    j = jj * 2 + b
                    c = wid + j * NW

                    @pl.when(c < NFULL)
                    def _():
                        @pl.when(c + NW < NFULL)
                        def _():
                            issue_in(c + NW, 1 - b)
                        wait_in(c, b)

                        @pl.when(j >= 2)
                        def _():
                            pltpu.make_async_copy(
                                *out_args(c - 2 * NW, b)).wait()
                        transpose_groups(ins[b], outs[b], TW // 16)
                        pltpu.async_copy(*out_args(c, b))
                return carry

            lax.fori_loop(0, (NCH + 1) // 2, pair_body, 0)

            for b_ in (0, 1):
                j_b = jmax - ((jmax - b_) % 2)

                @pl.when(j_b >= 0)
                def _():
                    pltpu.make_async_copy(
                        *out_args(wid + j_b * NW, b_)).wait()

            @pl.when(wid == NW - 1)
            def _():
                def tail_inner(t_v):
                    pltpu.sync_copy(tail, t_v)

                    def gbody(g, carry):
                        r0 = g * 16
                        for m in range(32):
                            dv = d_lo if m % 2 == 0 else d_hi
                            idx = dv * TAIL + (r0 + (m // 2))
                            v = plsc.load_gather(t_v, [idx])
                            out_v[pl.ds(r0 * EMB_DIM + m * 16, 16)] = v
                        return carry

                    lax.fori_loop(0, TAIL // 16, gbody, 0)
                    pltpu.sync_copy(
                        out_v.at[pl.ds(0, TAIL * EMB_DIM)],
                        dst.at[pl.ds(NFULL * TW * EMB_DIM, TAIL * EMB_DIM)])

                pl.run_scoped(
                    tail_inner,
                    pltpu.VMEM((EMB_DIM * TAIL,), jnp.float32))

    return k(ot_t, it_t, o_tail, i_tail)


def _sc_scores(ctx_idx, neg_idx, tgt_idx, i_table, o_table):
    mesh = plsc.VectorSubcoreMesh(core_axis_name="c", subcore_axis_name="s")

    @functools.partial(
        pl.kernel,
        mesh=mesh,
        compiler_params=pltpu.CompilerParams(
            use_tc_tiling_on_sc=False, needs_layout_passes=False),
        out_type=(
            jax.ShapeDtypeStruct((BATCH_N * CTX_LEN,), jnp.float32),
            jax.ShapeDtypeStruct((BATCH_N * NEG_K,), jnp.float32),
        ),
        scratch_types=[
            pltpu.VMEM((CROWS,), jnp.int32),
            pltpu.VMEM((CROWS,), jnp.int32),
            pltpu.VMEM((CB,), jnp.int32),
            pltpu.VMEM((CROWS, EMB_DIM), jnp.float32),
            pltpu.VMEM((CROWS, EMB_DIM), jnp.float32),
            pltpu.VMEM((CB, EMB_DIM), jnp.float32),
            pltpu.VMEM((CROWS,), jnp.float32),
            pltpu.VMEM((CROWS,), jnp.float32),
            pltpu.SemaphoreType.DMA,
        ],
    )
    def k(ctx_hbm, neg_hbm, tgt_hbm, it_hbm, ot_hbm, ps_hbm, ns_hbm,
          cidx_v, nidx_v, tidx_v, crows_v, nrows_v, trows_v, ps_v, ns_v, sem):
        wid = lax.axis_index("s") * NUM_CORES + lax.axis_index("c")
        lane = lax.iota(jnp.int32, 16)
        masks = [lane == l for l in range(16)]

        def chunk_body(t, carry0):
            roff = (wid * NCHUNK + t) * CROWS
            boff = (wid * NCHUNK + t) * CB
            pltpu.sync_copy(ctx_hbm.at[pl.ds(roff, CROWS)], cidx_v)
            pltpu.sync_copy(neg_hbm.at[pl.ds(roff, CROWS)], nidx_v)
            pltpu.sync_copy(tgt_hbm.at[pl.ds(boff, CB)], tidx_v)
            g1 = pltpu.async_copy(ot_hbm.at[cidx_v], crows_v, sem)
            g2 = pltpu.async_copy(ot_hbm.at[nidx_v], nrows_v, sem)
            g3 = pltpu.async_copy(it_hbm.at[tidx_v], trows_v, sem)
            g1.wait()
            g2.wait()
            g3.wait()

            def group_body(g, carry1):
                t0 = t1 = None
                pacc = nacc = None
                for j in range(4 * CTX_LEN):
                    if j % CTX_LEN == 0:
                        b = g * 4 + (j // CTX_LEN)
                        t0 = trows_v[b, pl.ds(0, 16)]
                        t1 = trows_v[b, pl.ds(16, 16)]
                    i = g * (4 * CTX_LEN) + j
                    v, l = j // 16, j % 16
                    p = (crows_v[i, pl.ds(0, 16)] * t0
                         + crows_v[i, pl.ds(16, 16)] * t1)
                    q = (nrows_v[i, pl.ds(0, 16)] * t0
                         + nrows_v[i, pl.ds(16, 16)] * t1)
                    ps = jnp.sum(p)
                    ns = jnp.sum(q)
                    if l == 0:
                        pacc = jnp.where(masks[0], ps, 0.0)
                        nacc = jnp.where(masks[0], ns, 0.0)
                    else:
                        pacc = jnp.where(masks[l], ps, pacc)
                        nacc = jnp.where(masks[l], ns, nacc)
                    if l == 15:
                        off = g * (4 * CTX_LEN) + v * 16
                        ps_v[pl.ds(off, 16)] = pacc
                        ns_v[pl.ds(off, 16)] = nacc
                return carry1

            lax.fori_loop(0, CROWS // (4 * CTX_LEN), group_body, 0)
            pltpu.sync_copy(ps_v, ps_hbm.at[pl.ds(roff, CROWS)])
            pltpu.sync_copy(ns_v, ns_hbm.at[pl.ds(roff, CROWS)])
            return carry0

        lax.fori_loop(0, NCHUNK, chunk_body, 0)

    return k(ctx_idx, neg_idx, tgt_idx, i_table, o_table)


def _log_sigmoid(x):
    return jnp.minimum(x, 0.0) - jnp.log1p(jnp.exp(-jnp.abs(x)))


def _tc_loss_body(ps_ref, ns_ref, out_ref):
    out_ref[0] = jnp.sum(_log_sigmoid(ps_ref[...]))
    out_ref[1] = jnp.sum(_log_sigmoid(-ns_ref[...]))


def _tc_loss(pos_s, neg_s):
    n = BATCH_N * CTX_LEN
    acc = pl.pallas_call(
        _tc_loss_body,
        in_specs=[
            pl.BlockSpec((n // 128, 128), lambda: (0, 0)),
            pl.BlockSpec((n // 128, 128), lambda: (0, 0)),
        ],
        out_specs=pl.BlockSpec(memory_space=pltpu.SMEM),
        out_shape=jax.ShapeDtypeStruct((2,), jnp.float32),
    )(pos_s.reshape(n // 128, 128), neg_s.reshape(n // 128, 128))
    return -(acc[0] / CTX_LEN + acc[1]) / BATCH_N


def kernel(context, target, i_table, o_table):
    b = context.shape[0]
    neg_samples = jax.random.randint(
        jax.random.key(12345), (b, NEG_K), 0, EMB_COUNT - 1)
    ctx_idx = context.astype(jnp.int32).reshape(-1)
    neg_idx = neg_samples.astype(jnp.int32).reshape(-1)
    tgt_idx = target.astype(jnp.int32)
    o_tail = o_table.T[:, NFULL * TW:].reshape(-1)
    i_tail = i_table.T[:, NFULL * TW:].reshape(-1)
    oflat, iflat = _sc_detile(o_table.T, i_table.T, o_tail, i_tail)
    o_c = oflat.reshape(EMB_COUNT, EMB_DIM)
    i_c = iflat.reshape(EMB_COUNT, EMB_DIM)
    pos_s, neg_s = _sc_scores(ctx_idx, neg_idx, tgt_idx, i_c, o_c)
    return _tc_loss(pos_s, neg_s)


# speedup vs baseline: 3.5411x; 2.2547x over previous
import functools

import jax
import jax.numpy as jnp
from jax import lax
from jax.experimental import pallas as pl
from jax.experimental.pallas import tpu as pltpu
from jax.experimental.pallas import tpu_sc as plsc

EMB_COUNT = 1000000
EMB_DIM = 32
NEG_K = 20
CTX_LEN = 20
BATCH_N = 16384

NUM_CORES = 2
NUM_SUBCORES = 16
NW = NUM_CORES * NUM_SUBCORES
BPW = BATCH_N // NW
CB = 64
NCHUNK = BPW // CB
CROWS = CB * CTX_LEN


TW = 768
NFULL = EMB_COUNT // TW
TAIL = EMB_COUNT - NFULL * TW
NCH = (NFULL + NW - 1) // NW


def _sc_detile(ot_t, it_t, o_tail, i_tail):
    mesh = plsc.VectorSubcoreMesh(core_axis_name="c", subcore_axis_name="s")

    @functools.partial(
        pl.kernel,
        mesh=mesh,
        compiler_params=pltpu.CompilerParams(
            use_tc_tiling_on_sc=True, needs_layout_passes=False),
        out_type=(
            jax.ShapeDtypeStruct((EMB_COUNT * EMB_DIM,), jnp.float32),
            jax.ShapeDtypeStruct((EMB_COUNT * EMB_DIM,), jnp.float32),
        ),
        scratch_types=[
            pltpu.VMEM((EMB_DIM, TW), jnp.float32),
            pltpu.VMEM((EMB_DIM, TW), jnp.float32),
            pltpu.VMEM((TW * EMB_DIM,), jnp.float32),
            pltpu.VMEM((TW * EMB_DIM,), jnp.float32),
            pltpu.SemaphoreType.DMA,
            pltpu.SemaphoreType.DMA,
            pltpu.SemaphoreType.DMA,
            pltpu.SemaphoreType.DMA,
        ],
    )
    def k(ot_hbm, it_hbm, otail_hbm, itail_hbm, oflat_hbm, iflat_hbm,
          in_v0, in_v1, out_v0, out_v1, s_i0, s_i1, s_o0, s_o1):
        wid = lax.axis_index("s") * NUM_CORES + lax.axis_index("c")
        lane = lax.iota(jnp.int32, 16)
        d_lo = lane
        d_hi = lane + 16
        ins = (in_v0, in_v1)
        outs = (out_v0, out_v1)
        s_in = (s_i0, s_i1)
        s_out = (s_o0, s_o1)
        out_v = out_v0

        lane32 = lane * EMB_DIM
        rots = [((lane + p) & 7) for p in range(8)]

        def transpose_groups(in_ref, out_ref, ngroups):
            def gbody(g, carry):
                r0 = g * 16
                ridx = lane + r0
                sbase = lane32 + r0 * EMB_DIM
                for kk in range(EMB_DIM // 8):
                    for p in range(8):
                        dv = rots[p] + (8 * kk)
                        v = plsc.load_gather(in_ref, [dv, ridx])
                        plsc.store_scatter(out_ref, [sbase + dv], v)
                return carry
            lax.fori_loop(0, ngroups, gbody, 0)

        jmax = (NFULL - 1 - wid) // NW

        for src, tail, dst in ((ot_hbm, otail_hbm, oflat_hbm),
                               (it_hbm, itail_hbm, iflat_hbm)):

            def in_args(c, b, kk):
                return (src.at[pl.ds(kk * 8, 8), pl.ds(c * TW, TW)],
                        ins[b].at[pl.ds(kk * 8, 8), :], s_in[b])

            def out_args(c, b):
                return (outs[b],
                        dst.at[pl.ds(c * TW * EMB_DIM, TW * EMB_DIM)],
                        s_out[b])

            def issue_in(c, b):
                for kk in range(EMB_DIM // 8):
                    pltpu.async_copy(*in_args(c, b, kk))

            def wait_in(c, b):
                for kk in range(EMB_DIM // 8):
                    pltpu.make_async_copy(*in_args(c, b, kk)).wait()

            issue_in(wid, 0)

            def pair_body(jj, carry):
                for b in (0, 1):
                    j = jj * 2 + b
                    c = wid + j * NW

                    @pl.when(c < NFULL)
                    def _():
                        @pl.when(c + NW < NFULL)
                        def _():
                            issue_in(c + NW, 1 - b)
                        wait_in(c, b)

                        @pl.when(j >= 2)
                        def _():
                            pltpu.make_async_copy(
                                *out_args(c - 2 * NW, b)).wait()
                        transpose_groups(ins[b], outs[b], TW // 16)
                        pltpu.async_copy(*out_args(c, b))
                return carry

            lax.fori_loop(0, (NCH + 1) // 2, pair_body, 0)

            for b_ in (0, 1):
                j_b = jmax - ((jmax - b_) % 2)

                @pl.when(j_b >= 0)
                def _():
                    pltpu.make_async_copy(
                        *out_args(wid + j_b * NW, b_)).wait()

            @pl.when(wid == NW - 1)
            def _():
                def tail_inner(t_v):
                    pltpu.sync_copy(tail, t_v)

                    def gbody(g, carry):
                        r0 = g * 16
                        for m in range(32):
                            dv = d_lo if m % 2 == 0 else d_hi
                            idx = dv * TAIL + (r0 + (m // 2))
                            v = plsc.load_gather(t_v, [idx])
                            out_v[pl.ds(r0 * EMB_DIM + m * 16, 16)] = v
                        return carry

                    lax.fori_loop(0, TAIL // 16, gbody, 0)
                    pltpu.sync_copy(
                        out_v.at[pl.ds(0, TAIL * EMB_DIM)],
                        dst.at[pl.ds(NFULL * TW * EMB_DIM, TAIL * EMB_DIM)])

                pl.run_scoped(
                    tail_inner,
                    pltpu.VMEM((EMB_DIM * TAIL,), jnp.float32))

    return k(ot_t, it_t, o_tail, i_tail)


def _sc_scores(ctx_idx, neg_idx, tgt_idx, i_table, o_table):
    mesh = plsc.VectorSubcoreMesh(core_axis_name="c", subcore_axis_name="s")

    @functools.partial(
        pl.kernel,
        mesh=mesh,
        compiler_params=pltpu.CompilerParams(
            use_tc_tiling_on_sc=False, needs_layout_passes=False),
        out_type=(
            jax.ShapeDtypeStruct((BATCH_N * CTX_LEN,), jnp.float32),
            jax.ShapeDtypeStruct((BATCH_N * NEG_K,), jnp.float32),
        ),
        scratch_types=[
            pltpu.VMEM((CROWS,), jnp.int32),
            pltpu.VMEM((CROWS,), jnp.int32),
            pltpu.VMEM((CB,), jnp.int32),
            pltpu.VMEM((CROWS, EMB_DIM), jnp.float32),
            pltpu.VMEM((CROWS, EMB_DIM), jnp.float32),
            pltpu.VMEM((CB, EMB_DIM), jnp.float32),
            pltpu.VMEM((CROWS,), jnp.float32),
            pltpu.VMEM((CROWS,), jnp.float32),
            pltpu.SemaphoreType.DMA,
        ],
    )
    def k(ctx_hbm, neg_hbm, tgt_hbm, it_hbm, ot_hbm, ps_hbm, ns_hbm,
          cidx_v, nidx_v, tidx_v, crows_v, nrows_v, trows_v, ps_v, ns_v, sem):
        wid = lax.axis_index("s") * NUM_CORES + lax.axis_index("c")
        lane = lax.iota(jnp.int32, 16)
        masks = [lane == l for l in range(16)]

        def chunk_body(t, carry0):
            roff = (wid * NCHUNK + t) * CROWS
            boff = (wid * NCHUNK + t) * CB
            pltpu.sync_copy(ctx_hbm.at[pl.ds(roff, CROWS)], cidx_v)
            pltpu.sync_copy(neg_hbm.at[pl.ds(roff, CROWS)], nidx_v)
            pltpu.sync_copy(tgt_hbm.at[pl.ds(boff, CB)], tidx_v)
            g1 = pltpu.async_copy(ot_hbm.at[cidx_v], crows_v, sem)
            g2 = pltpu.async_copy(ot_hbm.at[nidx_v], nrows_v, sem)
            g3 = pltpu.async_copy(it_hbm.at[tidx_v], trows_v, sem)
            g1.wait()
            g2.wait()
            g3.wait()

            def group_body(g, carry1):
                t0 = t1 = None
                pacc = nacc = None
                for j in range(4 * CTX_LEN):
                    if j % CTX_LEN == 0:
                        b = g * 4 + (j // CTX_LEN)
                        t0 = trows_v[b, pl.ds(0, 16)]
                        t1 = trows_v[b, pl.ds(16, 16)]
                    i = g * (4 * CTX_LEN) + j
                    v, l = j // 16, j % 16
                    p = (crows_v[i, pl.ds(0, 16)] * t0
                         + crows_v[i, pl.ds(16, 16)] * t1)
                    q = (nrows_v[i, pl.ds(0, 16)] * t0
                         + nrows_v[i, pl.ds(16, 16)] * t1)
                    ps = jnp.sum(p)
                    ns = jnp.sum(q)
                    if l == 0:
                        pacc = jnp.where(masks[0], ps, 0.0)
                        nacc = jnp.where(masks[0], ns, 0.0)
                    else:
                        pacc = jnp.where(masks[l], ps, pacc)
                        nacc = jnp.where(masks[l], ns, nacc)
                    if l == 15:
                        off = g * (4 * CTX_LEN) + v * 16
                        ps_v[pl.ds(off, 16)] = pacc
                        ns_v[pl.ds(off, 16)] = nacc
                return carry1

            lax.fori_loop(0, CROWS // (4 * CTX_LEN), group_body, 0)
            pltpu.sync_copy(ps_v, ps_hbm.at[pl.ds(roff, CROWS)])
            pltpu.sync_copy(ns_v, ns_hbm.at[pl.ds(roff, CROWS)])
            return carry0

        lax.fori_loop(0, NCHUNK, chunk_body, 0)

    return k(ctx_idx, neg_idx, tgt_idx, i_table, o_table)


def _log_sigmoid(x):
    return jnp.minimum(x, 0.0) - jnp.log1p(jnp.exp(-jnp.abs(x)))


def _tc_loss_body(ps_ref, ns_ref, out_ref):
    out_ref[0] = jnp.sum(_log_sigmoid(ps_ref[...]))
    out_ref[1] = jnp.sum(_log_sigmoid(-ns_ref[...]))


def _tc_loss(pos_s, neg_s):
    n = BATCH_N * CTX_LEN
    acc = pl.pallas_call(
        _tc_loss_body,
        in_specs=[
            pl.BlockSpec((n // 128, 128), lambda: (0, 0)),
            pl.BlockSpec((n // 128, 128), lambda: (0, 0)),
        ],
        out_specs=pl.BlockSpec(memory_space=pltpu.SMEM),
        out_shape=jax.ShapeDtypeStruct((2,), jnp.float32),
    )(pos_s.reshape(n // 128, 128), neg_s.reshape(n // 128, 128))
    return -(acc[0] / CTX_LEN + acc[1]) / BATCH_N


def kernel(context, target, i_table, o_table):
    b = context.shape[0]
    neg_samples = jax.random.randint(
        jax.random.key(12345), (b, NEG_K), 0, EMB_COUNT - 1)
    ctx_idx = context.astype(jnp.int32).reshape(-1)
    neg_idx = neg_samples.astype(jnp.int32).reshape(-1)
    tgt_idx = target.astype(jnp.int32)
    o_tail = o_table.T[:, NFULL * TW:].reshape(-1)
    i_tail = i_table.T[:, NFULL * TW:].reshape(-1)
    oflat, iflat = _sc_detile(o_table.T, i_table.T, o_tail, i_tail)
    o_c = oflat.reshape(EMB_COUNT, EMB_DIM)
    i_c = iflat.reshape(EMB_COUNT, EMB_DIM)
    pos_s, neg_s = _sc_scores(ctx_idx, neg_idx, tgt_idx, i_c, o_c)
    return _tc_loss(pos_s, neg_s)


# speedup vs baseline: 5.5495x; 1.5672x over previous
import functools

import jax
import jax.numpy as jnp
from jax import lax
from jax.experimental import pallas as pl
from jax.experimental.pallas import tpu as pltpu
from jax.experimental.pallas import tpu_sc as plsc

EMB_COUNT = 1000000
EMB_DIM = 32
NEG_K = 20
CTX_LEN = 20
BATCH_N = 16384

NUM_CORES = 2
NUM_SUBCORES = 16
NW = NUM_CORES * NUM_SUBCORES
BPW = BATCH_N // NW
CB = 64
NCHUNK = BPW // CB
CROWS = CB * CTX_LEN


TW = 768
NFULL = EMB_COUNT // TW
TAIL = EMB_COUNT - NFULL * TW
NCH = (NFULL + NW - 1) // NW


def _sc_detile(ot_t, o_tail):
    mesh = plsc.VectorSubcoreMesh(core_axis_name="c", subcore_axis_name="s")

    @functools.partial(
        pl.kernel,
        mesh=mesh,
        compiler_params=pltpu.CompilerParams(
            use_tc_tiling_on_sc=True, needs_layout_passes=False),
        out_type=jax.ShapeDtypeStruct((EMB_COUNT * EMB_DIM,), jnp.float32),
        scratch_types=[
            pltpu.VMEM((EMB_DIM, TW), jnp.float32),
            pltpu.VMEM((EMB_DIM, TW), jnp.float32),
            pltpu.VMEM((TW * EMB_DIM,), jnp.float32),
            pltpu.VMEM((TW * EMB_DIM,), jnp.float32),
            pltpu.SemaphoreType.DMA,
            pltpu.SemaphoreType.DMA,
            pltpu.SemaphoreType.DMA,
            pltpu.SemaphoreType.DMA,
        ],
    )
    def k(ot_hbm, otail_hbm, oflat_hbm,
          in_v0, in_v1, out_v0, out_v1, s_i0, s_i1, s_o0, s_o1):
        wid = lax.axis_index("s") * NUM_CORES + lax.axis_index("c")
        lane = lax.iota(jnp.int32, 16)
        d_lo = lane
        d_hi = lane + 16
        ins = (in_v0, in_v1)
        outs = (out_v0, out_v1)
        s_in = (s_i0, s_i1)
        s_out = (s_o0, s_o1)
        out_v = out_v0

        lane32 = lane * EMB_DIM
        rots = [((lane + p) & 7) for p in range(8)]

        def transpose_groups(in_ref, out_ref, ngroups):
            def gbody(g, carry):
                r0 = g * 16
                ridx = lane + r0
                sbase = lane32 + r0 * EMB_DIM
                for kk in range(EMB_DIM // 8):
                    for p in range(8):
                        dv = rots[p] + (8 * kk)
                        v = plsc.load_gather(in_ref, [dv, ridx])
                        plsc.store_scatter(out_ref, [sbase + dv], v)
                return carry
            lax.fori_loop(0, ngroups, gbody, 0)

        jmax = (NFULL - 1 - wid) // NW

        for src, tail, dst in ((ot_hbm, otail_hbm, oflat_hbm),):

            def in_args(c, b, kk):
                return (src.at[pl.ds(kk * 8, 8), pl.ds(c * TW, TW)],
                        ins[b].at[pl.ds(kk * 8, 8), :], s_in[b])

            def out_args(c, b):
                return (outs[b],
                        dst.at[pl.ds(c * TW * EMB_DIM, TW * EMB_DIM)],
                        s_out[b])

            def issue_in(c, b):
                for kk in range(EMB_DIM // 8):
                    pltpu.async_copy(*in_args(c, b, kk))

            def wait_in(c, b):
                for kk in range(EMB_DIM // 8):
                    pltpu.make_async_copy(*in_args(c, b, kk)).wait()

            issue_in(wid, 0)

            def pair_body(jj, carry):
                for b in (0, 1):
                    j = jj * 2 + b
                    c = wid + j * NW

                    @pl.when(c < NFULL)
                    def _():
                        @pl.when(c + NW < NFULL)
                        def _():
                            issue_in(c + NW, 1 - b)
                        wait_in(c, b)

                        @pl.when(j >= 2)
                        def _():
                            pltpu.make_async_copy(
                                *out_args(c - 2 * NW, b)).wait()
                        transpose_groups(ins[b], outs[b], TW // 16)
                        pltpu.async_copy(*out_args(c, b))
                return carry

            lax.fori_loop(0, (NCH + 1) // 2, pair_body, 0)

            for b_ in (0, 1):
                j_b = jmax - ((jmax - b_) % 2)

                @pl.when(j_b >= 0)
                def _():
                    pltpu.make_async_copy(
                        *out_args(wid + j_b * NW, b_)).wait()

            @pl.when(wid == NW - 1)
            def _():
                def tail_inner(t_v):
                    pltpu.sync_copy(tail, t_v)

                    def gbody(g, carry):
                        r0 = g * 16
                        for m in range(32):
                            dv = d_lo if m % 2 == 0 else d_hi
                            idx = dv * TAIL + (r0 + (m // 2))
                            v = plsc.load_gather(t_v, [idx])
                            out_v[pl.ds(r0 * EMB_DIM + m * 16, 16)] = v
                        return carry

                    lax.fori_loop(0, TAIL // 16, gbody, 0)
                    pltpu.sync_copy(
                        out_v.at[pl.ds(0, TAIL * EMB_DIM)],
                        dst.at[pl.ds(NFULL * TW * EMB_DIM, TAIL * EMB_DIM)])

                pl.run_scoped(
                    tail_inner,
                    pltpu.VMEM((EMB_DIM * TAIL,), jnp.float32))

    return k(ot_t, o_tail)


def _sc_scores(ctx_idx, neg_idx, tgt_emb_flat, o_table):
    mesh = plsc.VectorSubcoreMesh(core_axis_name="c", subcore_axis_name="s")

    @functools.partial(
        pl.kernel,
        mesh=mesh,
        compiler_params=pltpu.CompilerParams(
            use_tc_tiling_on_sc=False, needs_layout_passes=False),
        out_type=(
            jax.ShapeDtypeStruct((BATCH_N * CTX_LEN,), jnp.float32),
            jax.ShapeDtypeStruct((BATCH_N * NEG_K,), jnp.float32),
        ),
        scratch_types=[
            pltpu.VMEM((CROWS,), jnp.int32),
            pltpu.VMEM((CROWS,), jnp.int32),
            pltpu.VMEM((CROWS, EMB_DIM), jnp.float32),
            pltpu.VMEM((CROWS, EMB_DIM), jnp.float32),
            pltpu.VMEM((CB * EMB_DIM,), jnp.float32),
            pltpu.VMEM((CROWS,), jnp.float32),
            pltpu.VMEM((CROWS,), jnp.float32),
            pltpu.SemaphoreType.DMA,
        ],
    )
    def k(ctx_hbm, neg_hbm, tgtemb_hbm, ot_hbm, ps_hbm, ns_hbm,
          cidx_v, nidx_v, crows_v, nrows_v, trows_v, ps_v, ns_v, sem):
        wid = lax.axis_index("s") * NUM_CORES + lax.axis_index("c")
        lane = lax.iota(jnp.int32, 16)
        masks = [lane == l for l in range(16)]

        def chunk_body(t, carry0):
            roff = (wid * NCHUNK + t) * CROWS
            boff = (wid * NCHUNK + t) * CB
            pltpu.sync_copy(ctx_hbm.at[pl.ds(roff, CROWS)], cidx_v)
            pltpu.sync_copy(neg_hbm.at[pl.ds(roff, CROWS)], nidx_v)
            g1 = pltpu.async_copy(ot_hbm.at[cidx_v], crows_v, sem)
            g2 = pltpu.async_copy(ot_hbm.at[nidx_v], nrows_v, sem)
            g3 = pltpu.async_copy(
                tgtemb_hbm.at[pl.ds(boff * EMB_DIM, CB * EMB_DIM)],
                trows_v, sem)
            g1.wait()
            g2.wait()
            g3.wait()

            def group_body(g, carry1):
                t0 = t1 = None
                pacc = nacc = None
                for j in range(4 * CTX_LEN):
                    if j % CTX_LEN == 0:
                        b = g * 4 + (j // CTX_LEN)
                        t0 = trows_v[pl.ds(b * EMB_DIM, 16)]
                        t1 = trows_v[pl.ds(b * EMB_DIM + 16, 16)]
                    i = g * (4 * CTX_LEN) + j
                    v, l = j // 16, j % 16
                    p = (crows_v[i, pl.ds(0, 16)] * t0
                         + crows_v[i, pl.ds(16, 16)] * t1)
                    q = (nrows_v[i, pl.ds(0, 16)] * t0
                         + nrows_v[i, pl.ds(16, 16)] * t1)
                    ps = jnp.sum(p)
                    ns = jnp.sum(q)
                    if l == 0:
                        pacc = jnp.where(masks[0], ps, 0.0)
                        nacc = jnp.where(masks[0], ns, 0.0)
                    else:
                        pacc = jnp.where(masks[l], ps, pacc)
                        nacc = jnp.where(masks[l], ns, nacc)
                    if l == 15:
                        off = g * (4 * CTX_LEN) + v * 16
                        ps_v[pl.ds(off, 16)] = pacc
                        ns_v[pl.ds(off, 16)] = nacc
                return carry1

            lax.fori_loop(0, CROWS // (4 * CTX_LEN), group_body, 0)
            pltpu.sync_copy(ps_v, ps_hbm.at[pl.ds(roff, CROWS)])
            pltpu.sync_copy(ns_v, ns_hbm.at[pl.ds(roff, CROWS)])
            return carry0

        lax.fori_loop(0, NCHUNK, chunk_body, 0)

    return k(ctx_idx, neg_idx, tgt_emb_flat, o_table)


def _log_sigmoid(x):
    return jnp.minimum(x, 0.0) - jnp.log1p(jnp.exp(-jnp.abs(x)))


def _tc_loss_body(ps_ref, ns_ref, out_ref):
    out_ref[0] = jnp.sum(_log_sigmoid(ps_ref[...]))
    out_ref[1] = jnp.sum(_log_sigmoid(-ns_ref[...]))


def _tc_loss(pos_s, neg_s):
    n = BATCH_N * CTX_LEN
    acc = pl.pallas_call(
        _tc_loss_body,
        in_specs=[
            pl.BlockSpec((n // 128, 128), lambda: (0, 0)),
            pl.BlockSpec((n // 128, 128), lambda: (0, 0)),
        ],
        out_specs=pl.BlockSpec(memory_space=pltpu.SMEM),
        out_shape=jax.ShapeDtypeStruct((2,), jnp.float32),
    )(pos_s.reshape(n // 128, 128), neg_s.reshape(n // 128, 128))
    return -(acc[0] / CTX_LEN + acc[1]) / BATCH_N


def kernel(context, target, i_table, o_table):
    b = context.shape[0]
    neg_samples = jax.random.randint(
        jax.random.key(12345), (b, NEG_K), 0, EMB_COUNT - 1)
    ctx_idx = context.astype(jnp.int32).reshape(-1)
    neg_idx = neg_samples.astype(jnp.int32).reshape(-1)
    tgt_idx = target.astype(jnp.int32)
    o_tail = o_table.T[:, NFULL * TW:].reshape(-1)
    oflat = _sc_detile(o_table.T, o_tail)
    o_c = oflat.reshape(EMB_COUNT, EMB_DIM)
    tgt_emb_flat = jnp.take(i_table, tgt_idx, axis=0).reshape(-1)
    pos_s, neg_s = _sc_scores(ctx_idx, neg_idx, tgt_emb_flat, o_c)
    return _tc_loss(pos_s, neg_s)


# speedup vs baseline: 5.5765x; 1.0049x over previous
import functools

import jax
import jax.numpy as jnp
from jax import lax
from jax.experimental import pallas as pl
from jax.experimental.pallas import tpu as pltpu
from jax.experimental.pallas import tpu_sc as plsc

EMB_COUNT = 1000000
EMB_DIM = 32
NEG_K = 20
CTX_LEN = 20
BATCH_N = 16384

NUM_CORES = 2
NUM_SUBCORES = 16
NW = NUM_CORES * NUM_SUBCORES
BPW = BATCH_N // NW
CB = 64
NCHUNK = BPW // CB
CROWS = CB * CTX_LEN


TW = 896
NFULL = EMB_COUNT // TW
TAIL = EMB_COUNT - NFULL * TW
NCH = (NFULL + NW - 1) // NW


def _sc_detile(ot_t, o_tail):
    mesh = plsc.VectorSubcoreMesh(core_axis_name="c", subcore_axis_name="s")

    @functools.partial(
        pl.kernel,
        mesh=mesh,
        compiler_params=pltpu.CompilerParams(
            use_tc_tiling_on_sc=True, needs_layout_passes=False),
        out_type=jax.ShapeDtypeStruct((EMB_COUNT * EMB_DIM,), jnp.float32),
        scratch_types=[
            pltpu.VMEM((EMB_DIM, TW), jnp.float32),
            pltpu.VMEM((EMB_DIM, TW), jnp.float32),
            pltpu.VMEM((TW * EMB_DIM,), jnp.float32),
            pltpu.VMEM((TW * EMB_DIM,), jnp.float32),
            pltpu.SemaphoreType.DMA,
            pltpu.SemaphoreType.DMA,
            pltpu.SemaphoreType.DMA,
            pltpu.SemaphoreType.DMA,
        ],
    )
    def k(ot_hbm, otail_hbm, oflat_hbm,
          in_v0, in_v1, out_v0, out_v1, s_i0, s_i1, s_o0, s_o1):
        wid = lax.axis_index("s") * NUM_CORES + lax.axis_index("c")
        lane = lax.iota(jnp.int32, 16)
        d_lo = lane
        d_hi = lane + 16
        ins = (in_v0, in_v1)
        outs = (out_v0, out_v1)
        s_in = (s_i0, s_i1)
        s_out = (s_o0, s_o1)
        out_v = out_v0

        lane32 = lane * EMB_DIM
        rots = [((lane + p) & 7) for p in range(8)]

        def transpose_groups(in_ref, out_ref, ngroups):
            def gbody(g, carry):
                r0 = g * 16
                ridx = lane + r0
                sbase = lane32 + r0 * EMB_DIM
                for kk in range(EMB_DIM // 8):
                    for p in range(8):
                        dv = rots[p] + (8 * kk)
                        v = plsc.load_gather(in_ref, [dv, ridx])
                        plsc.store_scatter(out_ref, [sbase + dv], v)
                return carry
            lax.fori_loop(0, ngroups, gbody, 0)

        jmax = (NFULL - 1 - wid) // NW

        for src, tail, dst in ((ot_hbm, otail_hbm, oflat_hbm),):

            def in_args(c, b, kk):
                return (src.at[pl.ds(kk * 8, 8), pl.ds(c * TW, TW)],
                        ins[b].at[pl.ds(kk * 8, 8), :], s_in[b])

            def out_args(c, b):
                return (outs[b],
                        dst.at[pl.ds(c * TW * EMB_DIM, TW * EMB_DIM)],
                        s_out[b])

            def issue_in(c, b):
                for kk in range(EMB_DIM // 8):
                    pltpu.async_copy(*in_args(c, b, kk))

            def wait_in(c, b):
                for kk in range(EMB_DIM // 8):
                    pltpu.make_async_copy(*in_args(c, b, kk)).wait()

            issue_in(wid, 0)

            def pair_body(jj, carry):
                for b in (0, 1):
                    j = jj * 2 + b
                    c = wid + j * NW

                    @pl.when(c < NFULL)
                    def _():
                        @pl.when(c + NW < NFULL)
                        def _():
                            issue_in(c + NW, 1 - b)
                        wait_in(c, b)

                        @pl.when(j >= 2)
                        def _():
                            pltpu.make_async_copy(
                                *out_args(c - 2 * NW, b)).wait()
                        transpose_groups(ins[b], outs[b], TW // 16)
                        pltpu.async_copy(*out_args(c, b))
                return carry

            lax.fori_loop(0, (NCH + 1) // 2, pair_body, 0)

            for b_ in (0, 1):
                j_b = jmax - ((jmax - b_) % 2)

                @pl.when(j_b >= 0)
                def _():
                    pltpu.make_async_copy(
                        *out_args(wid + j_b * NW, b_)).wait()

            @pl.when(wid == NW - 1)
            def _():
                def tail_inner(t_v):
                    pltpu.sync_copy(tail, t_v)

                    def gbody(g, carry):
                        r0 = g * 16
                        for m in range(32):
                            dv = d_lo if m % 2 == 0 else d_hi
                            idx = dv * TAIL + (r0 + (m // 2))
                            v = plsc.load_gather(t_v, [idx])
                            out_v[pl.ds(r0 * EMB_DIM + m * 16, 16)] = v
                        return carry

                    lax.fori_loop(0, TAIL // 16, gbody, 0)
                    pltpu.sync_copy(
                        out_v.at[pl.ds(0, TAIL * EMB_DIM)],
                        dst.at[pl.ds(NFULL * TW * EMB_DIM, TAIL * EMB_DIM)])

                pl.run_scoped(
                    tail_inner,
                    pltpu.VMEM((EMB_DIM * TAIL,), jnp.float32))

    return k(ot_t, o_tail)


def _sc_scores(ctx_idx, neg_idx, tgt_emb_flat, o_table):
    mesh = plsc.VectorSubcoreMesh(core_axis_name="c", subcore_axis_name="s")

    @functools.partial(
        pl.kernel,
        mesh=mesh,
        compiler_params=pltpu.CompilerParams(
            use_tc_tiling_on_sc=False, needs_layout_passes=False),
        out_type=(
            jax.ShapeDtypeStruct((BATCH_N * CTX_LEN,), jnp.float32),
            jax.ShapeDtypeStruct((BATCH_N * NEG_K,), jnp.float32),
        ),
        scratch_types=[
            pltpu.VMEM((CROWS,), jnp.int32),
            pltpu.VMEM((CROWS,), jnp.int32),
            pltpu.VMEM((CROWS, EMB_DIM), jnp.float32),
            pltpu.VMEM((CROWS, EMB_DIM), jnp.float32),
            pltpu.VMEM((CB * EMB_DIM,), jnp.float32),
            pltpu.VMEM((CROWS,), jnp.float32),
            pltpu.VMEM((CROWS,), jnp.float32),
            pltpu.SemaphoreType.DMA,
        ],
    )
    def k(ctx_hbm, neg_hbm, tgtemb_hbm, ot_hbm, ps_hbm, ns_hbm,
          cidx_v, nidx_v, crows_v, nrows_v, trows_v, ps_v, ns_v, sem):
        wid = lax.axis_index("s") * NUM_CORES + lax.axis_index("c")
        lane = lax.iota(jnp.int32, 16)
        masks = [lane == l for l in range(16)]

        def chunk_body(t, carry0):
            roff = (wid * NCHUNK + t) * CROWS
            boff = (wid * NCHUNK + t) * CB
            pltpu.sync_copy(ctx_hbm.at[pl.ds(roff, CROWS)], cidx_v)
            pltpu.sync_copy(neg_hbm.at[pl.ds(roff, CROWS)], nidx_v)
            g1 = pltpu.async_copy(ot_hbm.at[cidx_v], crows_v, sem)
            g2 = pltpu.async_copy(ot_hbm.at[nidx_v], nrows_v, sem)
            g3 = pltpu.async_copy(
                tgtemb_hbm.at[pl.ds(boff * EMB_DIM, CB * EMB_DIM)],
                trows_v, sem)
            g1.wait()
            g2.wait()
            g3.wait()

            def group_body(g, carry1):
                t0 = t1 = None
                pacc = nacc = None
                for j in range(4 * CTX_LEN):
                    if j % CTX_LEN == 0:
                        b = g * 4 + (j // CTX_LEN)
                        t0 = trows_v[pl.ds(b * EMB_DIM, 16)]
                        t1 = trows_v[pl.ds(b * EMB_DIM + 16, 16)]
                    i = g * (4 * CTX_LEN) + j
                    v, l = j // 16, j % 16
                    p = (crows_v[i, pl.ds(0, 16)] * t0
                         + crows_v[i, pl.ds(16, 16)] * t1)
                    q = (nrows_v[i, pl.ds(0, 16)] * t0
                         + nrows_v[i, pl.ds(16, 16)] * t1)
                    ps = jnp.sum(p)
                    ns = jnp.sum(q)
                    if l == 0:
                        pacc = jnp.where(masks[0], ps, 0.0)
                        nacc = jnp.where(masks[0], ns, 0.0)
                    else:
                        pacc = jnp.where(masks[l], ps, pacc)
                        nacc = jnp.where(masks[l], ns, nacc)
                    if l == 15:
                        off = g * (4 * CTX_LEN) + v * 16
                        ps_v[pl.ds(off, 16)] = pacc
                        ns_v[pl.ds(off, 16)] = nacc
                return carry1

            lax.fori_loop(0, CROWS // (4 * CTX_LEN), group_body, 0)
            pltpu.sync_copy(ps_v, ps_hbm.at[pl.ds(roff, CROWS)])
            pltpu.sync_copy(ns_v, ns_hbm.at[pl.ds(roff, CROWS)])
            return carry0

        lax.fori_loop(0, NCHUNK, chunk_body, 0)

    return k(ctx_idx, neg_idx, tgt_emb_flat, o_table)


def _log_sigmoid(x):
    return jnp.minimum(x, 0.0) - jnp.log1p(jnp.exp(-jnp.abs(x)))


def _tc_loss_body(ps_ref, ns_ref, out_ref):
    out_ref[0] = jnp.sum(_log_sigmoid(ps_ref[...]))
    out_ref[1] = jnp.sum(_log_sigmoid(-ns_ref[...]))


def _tc_loss(pos_s, neg_s):
    n = BATCH_N * CTX_LEN
    acc = pl.pallas_call(
        _tc_loss_body,
        in_specs=[
            pl.BlockSpec((n // 128, 128), lambda: (0, 0)),
            pl.BlockSpec((n // 128, 128), lambda: (0, 0)),
        ],
        out_specs=pl.BlockSpec(memory_space=pltpu.SMEM),
        out_shape=jax.ShapeDtypeStruct((2,), jnp.float32),
    )(pos_s.reshape(n // 128, 128), neg_s.reshape(n // 128, 128))
    return -(acc[0] / CTX_LEN + acc[1]) / BATCH_N


def kernel(context, target, i_table, o_table):
    b = context.shape[0]
    neg_samples = jax.random.randint(
        jax.random.key(12345), (b, NEG_K), 0, EMB_COUNT - 1)
    ctx_idx = context.astype(jnp.int32).reshape(-1)
    neg_idx = neg_samples.astype(jnp.int32).reshape(-1)
    tgt_idx = target.astype(jnp.int32)
    o_tail = o_table.T[:, NFULL * TW:].reshape(-1)
    oflat = _sc_detile(o_table.T, o_tail)
    o_c = oflat.reshape(EMB_COUNT, EMB_DIM)
    tgt_emb_flat = jnp.take(i_table, tgt_idx, axis=0).reshape(-1)
    pos_s, neg_s = _sc_scores(ctx_idx, neg_idx, tgt_emb_flat, o_c)
    return _tc_loss(pos_s, neg_s)
